# Initial kernel scaffold; baseline (speedup 1.0000x reference)
import jax
import jax.numpy as jnp
from jax.experimental import pallas as pl


def kernel(x, edge_index, i, W1, b1, W2, b2, W3, b3, Wd1, bd1, Wd2, bd2):
    raise NotImplementedError("write your pallas kernel here")


# speedup vs baseline: 23.9036x; 23.9036x over previous
import functools

import jax
import jax.numpy as jnp
from jax import lax
from jax.experimental import pallas as pl
from jax.experimental.pallas import tpu as pltpu
from jax.experimental.pallas import tpu_sc as plsc

_N = 10000
_E = 320000
_F = 128
_H = 32
_G = 64

_NC = 2
_NS = 16
_NW = _NC * _NS
_EPT = _E // _NW
_C = 80
_NCH = _EPT // _C
_NP = 10240
_RPS = _NP // _NS

_sc_mesh = plsc.VectorSubcoreMesh(core_axis_name="c", subcore_axis_name="s")
_sc_params = pltpu.CompilerParams(use_tc_tiling_on_sc=False)


@functools.partial(
    pl.kernel,
    out_type=jax.ShapeDtypeStruct((_NC, _NP, 16), jnp.float32),
    mesh=_sc_mesh,
    compiler_params=_sc_params,
    scratch_types=[
        pltpu.VMEM_SHARED((_NP, 16), jnp.float32),
        pltpu.VMEM((_NCH, _C), jnp.int32),
        pltpu.VMEM((_C, 16), jnp.float32),
        pltpu.VMEM((_RPS, 16), jnp.float32),
    ],
)
def _sc_degree(dst_hbm, ones_hbm, out_hbm, deg_sh, dst_v, ones_v, stage_v):
    c = lax.axis_index("c")
    s = lax.axis_index("s")
    wid = c * _NS + s

    zero16 = jnp.zeros((16,), jnp.float32)

    def zbody(r, carry):
        stage_v[r, :] = zero16
        return carry

    lax.fori_loop(0, _RPS, zbody, 0)
    pltpu.sync_copy(stage_v, deg_sh.at[pl.ds(s * _RPS, _RPS)])
    pltpu.sync_copy(dst_hbm.at[wid], dst_v)
    pltpu.sync_copy(ones_hbm, ones_v)
    plsc.subcore_barrier()

    def ebody(j, carry):
        pltpu.sync_copy(ones_v, deg_sh.at[dst_v.at[j]], add=True)
        return carry

    lax.fori_loop(0, _NCH, ebody, 0)
    plsc.subcore_barrier()

    pltpu.sync_copy(deg_sh.at[pl.ds(s * _RPS, _RPS)], stage_v)
    pltpu.sync_copy(stage_v, out_hbm.at[c, pl.ds(s * _RPS, _RPS)])


@functools.partial(
    pl.kernel,
    out_type=jax.ShapeDtypeStruct((_NC, _NP, _H), jnp.float32),
    mesh=_sc_mesh,
    compiler_params=_sc_params,
    scratch_types=[
        pltpu.VMEM_SHARED((_NP, _H), jnp.float32),
        pltpu.VMEM((_EPT,), jnp.int32),
        pltpu.VMEM((_NCH, _C), jnp.int32),
        pltpu.VMEM((_C, _H), jnp.float32),
        pltpu.VMEM((_RPS, _H), jnp.float32),
        pltpu.SemaphoreType.DMA,
    ],
)
def _sc_scatter(h_hbm, src_hbm, dst_hbm, out_hbm, acc_sh, src_v, dst_v,
                rows_v, stage_v, sem):
    c = lax.axis_index("c")
    s = lax.axis_index("s")
    wid = c * _NS + s

    zero16 = jnp.zeros((16,), jnp.float32)

    def zbody(r, carry):
        stage_v[r, pl.ds(0, 16)] = zero16
        stage_v[r, pl.ds(16, 16)] = zero16
        return carry

    lax.fori_loop(0, _RPS, zbody, 0)
    pltpu.sync_copy(stage_v, acc_sh.at[pl.ds(s * _RPS, _RPS)])
    pltpu.sync_copy(src_hbm.at[pl.ds(wid * _EPT, _EPT)], src_v)
    pltpu.sync_copy(dst_hbm.at[wid], dst_v)
    plsc.subcore_barrier()

    def ebody(j, carry):
        pltpu.async_copy(h_hbm.at[src_v.at[pl.ds(j * _C, _C)]], rows_v,
                         sem).wait()
        pltpu.sync_copy(rows_v, acc_sh.at[dst_v.at[j]], add=True)
        return carry

    lax.fori_loop(0, _NCH, ebody, 0)
    plsc.subcore_barrier()

    pltpu.sync_copy(acc_sh.at[pl.ds(s * _RPS, _RPS)], stage_v)
    pltpu.sync_copy(stage_v, out_hbm.at[c, pl.ds(s * _RPS, _RPS)])


def _tc_a_body(x_ref, w1_ref, deg_ref, inv_ref, h1p_ref):
    deg = deg_ref[0, 0:_N, 0:1] + deg_ref[1, 0:_N, 0:1] + 1.0
    inv = lax.rsqrt(jnp.maximum(deg, 1.0))
    inv_ref[...] = inv
    h = jnp.dot(x_ref[...], w1_ref[...], preferred_element_type=jnp.float32)
    h1p_ref[...] = h * inv


def _tc_a(x, W1, degp):
    return pl.pallas_call(
        _tc_a_body,
        out_shape=(
            jax.ShapeDtypeStruct((_N, 1), jnp.float32),
            jax.ShapeDtypeStruct((_N, _H), jnp.float32),
        ),
    )(x, W1, degp)


def _tc_b_body(acc_ref, hp_ref, inv_ref, b_ref, w_ref, out_ref):
    ssum = acc_ref[0, 0:_N, :] + acc_ref[1, 0:_N, :] + hp_ref[...]
    a = jnp.maximum(ssum * inv_ref[...] + b_ref[...], 0.0)
    h = jnp.dot(a, w_ref[...], preferred_element_type=jnp.float32)
    out_ref[...] = h * inv_ref[...]


def _tc_b(acc, hp, inv, b, Wn):
    return pl.pallas_call(
        _tc_b_body,
        out_shape=jax.ShapeDtypeStruct((_N, _H), jnp.float32),
    )(acc, hp, inv, b, Wn)


def _tc_c_body(acc_ref, hp_ref, inv_ref, b3_ref, i_ref, wd1_ref, bd1_ref,
               wd2_ref, bd2_ref, out_ref):
    ssum = acc_ref[0, 0:_N, :] + acc_ref[1, 0:_N, :] + hp_ref[...]
    a = jnp.maximum(ssum * inv_ref[...] + b3_ref[...], 0.0)
    gid = lax.broadcasted_iota(jnp.int32, (_G, _N), 0)
    onehot_t = jnp.where(i_ref[...] == gid, 1.0, 0.0)
    pooled = jnp.dot(onehot_t, a, preferred_element_type=jnp.float32)
    d1 = jnp.maximum(
        jnp.dot(pooled, wd1_ref[...], preferred_element_type=jnp.float32)
        + bd1_ref[...], 0.0)
    o = jnp.dot(d1, wd2_ref[...], preferred_element_type=jnp.float32) \
        + bd2_ref[...]
    out_ref[...] = 1.0 / (1.0 + jnp.exp(-o))


def _tc_c(acc, hp, inv, b3, iv, Wd1, bd1, Wd2, bd2):
    return pl.pallas_call(
        _tc_c_body,
        out_shape=jax.ShapeDtypeStruct((_G, 1), jnp.float32),
    )(acc, hp, inv, b3, iv, Wd1, bd1, Wd2, bd2)


def kernel(x, edge_index, i, W1, b1, W2, b2, W3, b3, Wd1, bd1, Wd2, bd2):
    src = edge_index[0]
    dst3 = edge_index[1].reshape(_NW, _NCH, _C)
    ones_c16 = jnp.ones((_C, 16), jnp.float32)

    degp = _sc_degree(dst3, ones_c16)
    inv, h1p = _tc_a(x, W1, degp)
    acc1 = _sc_scatter(h1p, src, dst3)
    h2p = _tc_b(acc1, h1p, inv, b1.reshape(1, _H), W2)
    acc2 = _sc_scatter(h2p, src, dst3)
    h3p = _tc_b(acc2, h2p, inv, b2.reshape(1, _H), W3)
    acc3 = _sc_scatter(h3p, src, dst3)
    out = _tc_c(acc3, h3p, inv, b3.reshape(1, _H), i.reshape(1, _N),
                Wd1, bd1.reshape(1, 26), Wd2, bd2.reshape(1, 1))
    return out


# speedup vs baseline: 27.7187x; 1.1596x over previous
import functools

import jax
import jax.numpy as jnp
from jax import lax
from jax.experimental import pallas as pl
from jax.experimental.pallas import tpu as pltpu
from jax.experimental.pallas import tpu_sc as plsc

_N = 10000
_E = 320000
_F = 128
_H = 32
_G = 64

_NC = 2
_NS = 16
_NW = _NC * _NS
_EPT = _E // _NW
_C = 80
_NCH = _EPT // _C
_NP = 10240
_RPS = _NP // _NS

_sc_mesh = plsc.VectorSubcoreMesh(core_axis_name="c", subcore_axis_name="s")
_sc_params = pltpu.CompilerParams(use_tc_tiling_on_sc=False)


@functools.partial(
    pl.kernel,
    out_type=jax.ShapeDtypeStruct((_NC, _NP, 16), jnp.float32),
    mesh=_sc_mesh,
    compiler_params=_sc_params,
    scratch_types=[
        pltpu.VMEM_SHARED((_NP, 16), jnp.float32),
        pltpu.VMEM((_NCH, _C), jnp.int32),
        pltpu.VMEM((_C, 16), jnp.float32),
        pltpu.SemaphoreType.DMA,
    ],
)
def _sc_degree(dst_hbm, ones_hbm, zeros_hbm, out_hbm, deg_sh, dst_v, ones_v,
               sem):
    c = lax.axis_index("c")
    s = lax.axis_index("s")
    wid = c * _NS + s

    pltpu.sync_copy(zeros_hbm.at[pl.ds(s * _RPS, _RPS)],
                    deg_sh.at[pl.ds(s * _RPS, _RPS)])
    pltpu.sync_copy(dst_hbm.at[wid], dst_v)
    pltpu.sync_copy(ones_hbm, ones_v)
    plsc.subcore_barrier()

    def ebody(j, carry):
        pltpu.async_copy(ones_v, deg_sh.at[dst_v.at[j]], sem, add=True)
        return carry

    lax.fori_loop(0, _NCH, ebody, 0)

    def dbody(j, carry):
        pltpu.make_async_copy(ones_hbm, ones_v, sem).wait()
        return carry

    lax.fori_loop(0, _NCH, dbody, 0)
    plsc.subcore_barrier()

    pltpu.sync_copy(deg_sh.at[pl.ds(s * _RPS, _RPS)],
                    out_hbm.at[c, pl.ds(s * _RPS, _RPS)])


@functools.partial(
    pl.kernel,
    out_type=jax.ShapeDtypeStruct((_NC, _NP, _H), jnp.float32),
    mesh=_sc_mesh,
    compiler_params=_sc_params,
    scratch_types=[
        pltpu.VMEM_SHARED((_NP, _H), jnp.float32),
        pltpu.VMEM((_EPT,), jnp.int32),
        pltpu.VMEM((_NCH, _C), jnp.int32),
        pltpu.VMEM((_C, _H), jnp.float32),
        pltpu.VMEM((_C, _H), jnp.float32),
        pltpu.SemaphoreType.DMA,
        pltpu.SemaphoreType.DMA,
    ],
)
def _sc_scatter(h_hbm, src_hbm, dst_hbm, zeros_hbm, out_hbm, acc_sh, src_v,
                dst_v, rows0_v, rows1_v, sem0, sem1):
    c = lax.axis_index("c")
    s = lax.axis_index("s")
    wid = c * _NS + s

    pltpu.sync_copy(zeros_hbm.at[pl.ds(s * _RPS, _RPS)],
                    acc_sh.at[pl.ds(s * _RPS, _RPS)])
    pltpu.sync_copy(src_hbm.at[pl.ds(wid * _EPT, _EPT)], src_v)
    pltpu.sync_copy(dst_hbm.at[wid], dst_v)
    plsc.subcore_barrier()

    pltpu.async_copy(h_hbm.at[src_v.at[pl.ds(0, _C)]], rows0_v, sem0)

    def ebody(jj, carry):
        j0 = 2 * jj
        pltpu.make_async_copy(h_hbm.at[pl.ds(0, _C)], rows0_v, sem0).wait()
        pltpu.async_copy(h_hbm.at[src_v.at[pl.ds((j0 + 1) * _C, _C)]],
                         rows1_v, sem1)
        pltpu.sync_copy(rows0_v, acc_sh.at[dst_v.at[j0]], add=True)
        pltpu.make_async_copy(h_hbm.at[pl.ds(0, _C)], rows1_v, sem1).wait()
        pltpu.async_copy(h_hbm.at[src_v.at[pl.ds((j0 + 2) * _C, _C)]],
                         rows0_v, sem0)
        pltpu.sync_copy(rows1_v, acc_sh.at[dst_v.at[j0 + 1]], add=True)
        return carry

    lax.fori_loop(0, (_NCH - 1) // 2, ebody, 0)
    pltpu.make_async_copy(h_hbm.at[pl.ds(0, _C)], rows0_v, sem0).wait()
    pltpu.sync_copy(rows0_v, acc_sh.at[dst_v.at[_NCH - 1]], add=True)
    plsc.subcore_barrier()

    pltpu.sync_copy(acc_sh.at[pl.ds(s * _RPS, _RPS)],
                    out_hbm.at[c, pl.ds(s * _RPS, _RPS)])


def _tc_a_body(x_ref, w1_ref, deg_ref, inv_ref, h1p_ref):
    deg = deg_ref[0, 0:_N, 0:1] + deg_ref[1, 0:_N, 0:1] + 1.0
    inv = lax.rsqrt(jnp.maximum(deg, 1.0))
    inv_ref[...] = inv
    h = jnp.dot(x_ref[...], w1_ref[...], preferred_element_type=jnp.float32)
    h1p_ref[...] = h * inv


def _tc_a(x, W1, degp):
    return pl.pallas_call(
        _tc_a_body,
        out_shape=(
            jax.ShapeDtypeStruct((_N, 1), jnp.float32),
            jax.ShapeDtypeStruct((_N, _H), jnp.float32),
        ),
    )(x, W1, degp)


def _tc_b_body(acc_ref, hp_ref, inv_ref, b_ref, w_ref, out_ref):
    ssum = acc_ref[0, 0:_N, :] + acc_ref[1, 0:_N, :] + hp_ref[...]
    a = jnp.maximum(ssum * inv_ref[...] + b_ref[...], 0.0)
    h = jnp.dot(a, w_ref[...], preferred_element_type=jnp.float32)
    out_ref[...] = h * inv_ref[...]


def _tc_b(acc, hp, inv, b, Wn):
    return pl.pallas_call(
        _tc_b_body,
        out_shape=jax.ShapeDtypeStruct((_N, _H), jnp.float32),
    )(acc, hp, inv, b, Wn)


def _tc_c_body(acc_ref, hp_ref, inv_ref, b3_ref, i_ref, wd1_ref, bd1_ref,
               wd2_ref, bd2_ref, out_ref):
    ssum = acc_ref[0, 0:_N, :] + acc_ref[1, 0:_N, :] + hp_ref[...]
    a = jnp.maximum(ssum * inv_ref[...] + b3_ref[...], 0.0)
    gid = lax.broadcasted_iota(jnp.int32, (_G, _N), 0)
    onehot_t = jnp.where(i_ref[...] == gid, 1.0, 0.0)
    pooled = jnp.dot(onehot_t, a, preferred_element_type=jnp.float32)
    d1 = jnp.maximum(
        jnp.dot(pooled, wd1_ref[...], preferred_element_type=jnp.float32)
        + bd1_ref[...], 0.0)
    o = jnp.dot(d1, wd2_ref[...], preferred_element_type=jnp.float32) \
        + bd2_ref[...]
    out_ref[...] = 1.0 / (1.0 + jnp.exp(-o))


def _tc_c(acc, hp, inv, b3, iv, Wd1, bd1, Wd2, bd2):
    return pl.pallas_call(
        _tc_c_body,
        out_shape=jax.ShapeDtypeStruct((_G, 1), jnp.float32),
    )(acc, hp, inv, b3, iv, Wd1, bd1, Wd2, bd2)


def kernel(x, edge_index, i, W1, b1, W2, b2, W3, b3, Wd1, bd1, Wd2, bd2):
    src = edge_index[0]
    dst3 = edge_index[1].reshape(_NW, _NCH, _C)
    ones_c16 = jnp.ones((_C, 16), jnp.float32)
    z16 = jnp.zeros((_NP, 16), jnp.float32)
    z32 = jnp.zeros((_NP, _H), jnp.float32)

    degp = _sc_degree(dst3, ones_c16, z16)
    inv, h1p = _tc_a(x, W1, degp)
    acc1 = _sc_scatter(h1p, src, dst3, z32)
    h2p = _tc_b(acc1, h1p, inv, b1.reshape(1, _H), W2)
    acc2 = _sc_scatter(h2p, src, dst3, z32)
    h3p = _tc_b(acc2, h2p, inv, b2.reshape(1, _H), W3)
    acc3 = _sc_scatter(h3p, src, dst3, z32)
    out = _tc_c(acc3, h3p, inv, b3.reshape(1, _H), i.reshape(1, _N),
                Wd1, bd1.reshape(1, 26), Wd2, bd2.reshape(1, 1))
    return out
